# baseline (speedup 1.0000x reference)
import jax
import jax.numpy as jnp
from jax import lax
from jax.experimental import pallas as pl
from jax.experimental.pallas import tpu as pltpu

_B = 16
_N = 4096


def _body(ui_ref, call_ref,
          W1_ref, b1_ref, W2_ref, b2_ref, Wr_ref, br_ref, Wt_ref, bt_ref,
          Wg1_ref, bg1_ref, Wo1_ref, bo1_ref, Wa1_ref, ba1_ref,
          Wg2_ref, bg2_ref, Wgs_ref, bgs_ref,
          Wo2_ref, bo2_ref, Wo3_ref, bo3_ref,
          Wa2_ref, ba2_ref, Wa3_ref, ba3_ref,
          wts_ref, sdf_ref, img_ref, aud_ref,
          tbg_ref, tbo_ref, tba_ref,
          Wg1T_ref, Wo1T_ref, Wa1T_ref,
          Wg2T_ref, Wo2T_ref, Wa2T_ref,
          WgsT_ref, Wo3T_ref, Wa3T_ref):
    b = pl.program_id(0)
    f32 = jnp.float32

    def dot(x, y):
        return jnp.dot(x, y, preferred_element_type=f32)


    @pl.when(b == 0)
    def _router():
        hs = ui_ref[...]
        for _ in range(3):
            m = jnp.tanh(dot(hs, W1_ref[...]) + b1_ref[...])
            hs = hs + jnp.tanh(dot(m, W2_ref[...]) + b2_ref[...])
        logits = dot(hs, Wr_ref[...]) + br_ref[...]
        wts_ref[...] = jax.nn.softmax(logits, axis=-1)
        thought = jnp.tanh(dot(hs, Wt_ref[...]) + bt_ref[...])
        tbg_ref[...] = dot(thought, Wg1_ref[3:, :]) + bg1_ref[...]
        tbo_ref[...] = dot(thought, Wo1_ref[2:, :]) + bo1_ref[...]
        tba_ref[...] = dot(thought, Wa1_ref[1:, :]) + ba1_ref[...]
        Wg1T_ref[...] = jnp.transpose(Wg1_ref[0:3, :])
        Wo1T_ref[...] = jnp.transpose(Wo1_ref[0:2, :])
        Wa1T_ref[...] = jnp.transpose(Wa1_ref[0:1, :])
        Wg2T_ref[...] = jnp.transpose(Wg2_ref[...])
        Wo2T_ref[...] = jnp.transpose(Wo2_ref[...])
        Wa2T_ref[...] = jnp.transpose(Wa2_ref[...])
        WgsT_ref[...] = jnp.transpose(Wgs_ref[...])
        Wo3T_ref[...] = jnp.transpose(Wo3_ref[...])
        Wa3T_ref[...] = jnp.transpose(Wa3_ref[...])

    w_row = wts_ref[pl.ds(b, 1), :]
    col = lambda v: jnp.transpose(v)

    h = jnp.maximum(dot(Wg1T_ref[...], call_ref[0][0:3])
                    + col(tbg_ref[pl.ds(b, 1), :]), 0.0)
    h = jnp.maximum(dot(Wg2T_ref[...], h) + col(bg2_ref[...]), 0.0)
    sdf = dot(WgsT_ref[...], h) + bgs_ref[...]
    sdf_ref[0] = sdf * w_row[:, 0:1]

    h = jnp.maximum(dot(Wo1T_ref[...], call_ref[0][3:5])
                    + col(tbo_ref[pl.ds(b, 1), :]), 0.0)
    h = jnp.maximum(dot(Wo2T_ref[...], h) + col(bo2_ref[...]), 0.0)
    img = dot(Wo3T_ref[...], h) + col(bo3_ref[...])
    img_ref[0] = jax.nn.sigmoid(img) * w_row[:, 1:2]

    h = jnp.maximum(Wa1T_ref[...] * call_ref[0][5:6] + col(tba_ref[pl.ds(b, 1), :]), 0.0)
    h = jnp.maximum(dot(Wa2T_ref[...], h) + col(ba2_ref[...]), 0.0)
    aud = jnp.tanh(dot(Wa3T_ref[...], h) + ba3_ref[...])
    aud_ref[0] = aud * w_row[:, 2:3]


def _full(shape):
    return pl.BlockSpec(shape, lambda b: (0,) * len(shape))


@jax.jit
def kernel(user_intent, coords_3d, coords_2d, coords_1d, W1, b1, W2, b2, Wr,
           br, Wt, bt, Wg1, bg1, Wg2, bg2, Wgs, bgs, Wgc, bgc, Wo1, bo1, Wo2,
           bo2, Wo3, bo3, Wa1, ba1, Wa2, ba2, Wa3, ba3):
    del Wgc, bgc
    B, N = _B, _N
    row = lambda v: v.reshape(1, -1)

    in_specs = [
        _full((B, 64)),
        pl.BlockSpec((1, 6, N), lambda b: (b, 0, 0)),
        _full((64, 64)), _full((1, 64)), _full((64, 64)), _full((1, 64)),
        _full((64, 3)), _full((1, 3)), _full((64, 16)), _full((1, 16)),
        _full((19, 256)), _full((1, 256)),
        _full((18, 256)), _full((1, 256)),
        _full((17, 256)), _full((1, 256)),
        _full((256, 256)), _full((1, 256)), _full((256, 1)), _full((1, 1)),
        _full((256, 256)), _full((1, 256)), _full((256, 3)), _full((1, 3)),
        _full((256, 256)), _full((1, 256)), _full((256, 1)), _full((1, 1)),
    ]
    out_specs = [
        _full((B, 3)),
        pl.BlockSpec((1, 1, N), lambda b: (b, 0, 0)),
        pl.BlockSpec((1, 3, N), lambda b: (b, 0, 0)),
        pl.BlockSpec((1, 1, N), lambda b: (b, 0, 0)),
    ]
    out_shapes = [
        jax.ShapeDtypeStruct((B, 3), jnp.float32),
        jax.ShapeDtypeStruct((B, 1, N), jnp.float32),
        jax.ShapeDtypeStruct((B, 3, N), jnp.float32),
        jax.ShapeDtypeStruct((B, 1, N), jnp.float32),
    ]

    weights, sdf_t, img_t, aud_t = pl.pallas_call(
        _body,
        grid=(B,),
        in_specs=in_specs,
        out_specs=out_specs,
        out_shape=out_shapes,
        scratch_shapes=[
            pltpu.VMEM((B, 256), jnp.float32),
            pltpu.VMEM((B, 256), jnp.float32),
            pltpu.VMEM((B, 256), jnp.float32),
            pltpu.VMEM((256, 3), jnp.float32),
            pltpu.VMEM((256, 2), jnp.float32),
            pltpu.VMEM((256, 1), jnp.float32),
            pltpu.VMEM((256, 256), jnp.float32),
            pltpu.VMEM((256, 256), jnp.float32),
            pltpu.VMEM((256, 256), jnp.float32),
            pltpu.VMEM((1, 256), jnp.float32),
            pltpu.VMEM((3, 256), jnp.float32),
            pltpu.VMEM((1, 256), jnp.float32),
        ],
    )(user_intent,
      jnp.concatenate([coords_3d, coords_2d, coords_1d],
                      axis=2).transpose(0, 2, 1),
      W1, row(b1), W2, row(b2), Wr, row(br), Wt, row(bt),
      Wg1, row(bg1), Wo1, row(bo1), Wa1, row(ba1),
      Wg2, row(bg2), Wgs, row(bgs),
      Wo2, row(bo2), Wo3, row(bo3),
      Wa2, row(ba2), Wa3, row(ba3))

    raw_sdf = sdf_t.reshape(B, N, 1)
    raw_img = img_t.transpose(0, 2, 1)
    raw_audio = aud_t.reshape(B, N, 1)
    return weights, raw_sdf, raw_img, raw_audio


# speedup vs baseline: 1.0483x; 1.0483x over previous
import jax
import jax.numpy as jnp
from jax import lax
from jax.experimental import pallas as pl
from jax.experimental.pallas import tpu as pltpu

_B = 16
_N = 4096


def _body(ui_ref, c3_ref, c2_ref, c1_ref,
          W1_ref, b1_ref, W2_ref, b2_ref, Wr_ref, br_ref, Wt_ref, bt_ref,
          Wg1_ref, bg1_ref, Wo1_ref, bo1_ref, Wa1_ref, ba1_ref,
          Wg2_ref, bg2_ref, Wgs_ref, bgs_ref,
          Wo2_ref, bo2_ref, Wo3_ref, bo3_ref,
          Wa2_ref, ba2_ref, Wa3_ref, ba3_ref,
          wts_ref, sdf_ref, img_ref, aud_ref,
          tbg_ref, tbo_ref, tba_ref,
          Wg1T_ref, Wo1T_ref, Wa1T_ref,
          Wg2T_ref, Wo2T_ref, Wa2T_ref,
          WgsT_ref, Wo3T_ref, Wa3T_ref):
    b = pl.program_id(0)
    f32 = jnp.float32

    def dot(x, y):
        return jnp.dot(x, y, preferred_element_type=f32)


    @pl.when(b == 0)
    def _router():
        hs = ui_ref[...]
        for _ in range(3):
            m = jnp.tanh(dot(hs, W1_ref[...]) + b1_ref[...])
            hs = hs + jnp.tanh(dot(m, W2_ref[...]) + b2_ref[...])
        logits = dot(hs, Wr_ref[...]) + br_ref[...]
        wts_ref[...] = jax.nn.softmax(logits, axis=-1)
        thought = jnp.tanh(dot(hs, Wt_ref[...]) + bt_ref[...])
        tbg_ref[...] = dot(thought, Wg1_ref[3:, :]) + bg1_ref[...]
        tbo_ref[...] = dot(thought, Wo1_ref[2:, :]) + bo1_ref[...]
        tba_ref[...] = dot(thought, Wa1_ref[1:, :]) + ba1_ref[...]
        Wg1T_ref[...] = jnp.transpose(Wg1_ref[0:3, :])
        Wo1T_ref[...] = jnp.transpose(Wo1_ref[0:2, :])
        Wa1T_ref[...] = jnp.transpose(Wa1_ref[0:1, :])
        Wg2T_ref[...] = jnp.transpose(Wg2_ref[...])
        Wo2T_ref[...] = jnp.transpose(Wo2_ref[...])
        Wa2T_ref[...] = jnp.transpose(Wa2_ref[...])
        WgsT_ref[...] = jnp.transpose(Wgs_ref[...])
        Wo3T_ref[...] = jnp.transpose(Wo3_ref[...])
        Wa3T_ref[...] = jnp.transpose(Wa3_ref[...])

    w_row = wts_ref[pl.ds(b, 1), :]
    col = lambda v: jnp.transpose(v)

    h = jnp.maximum(dot(Wg1T_ref[...], c3_ref[0])
                    + col(tbg_ref[pl.ds(b, 1), :]), 0.0)
    h = jnp.maximum(dot(Wg2T_ref[...], h) + col(bg2_ref[...]), 0.0)
    sdf = dot(WgsT_ref[...], h) + bgs_ref[...]
    sdf_ref[0] = sdf * w_row[:, 0:1]

    h = jnp.maximum(dot(Wo1T_ref[...], c2_ref[0])
                    + col(tbo_ref[pl.ds(b, 1), :]), 0.0)
    h = jnp.maximum(dot(Wo2T_ref[...], h) + col(bo2_ref[...]), 0.0)
    img = dot(Wo3T_ref[...], h) + col(bo3_ref[...])
    img_ref[0] = jax.nn.sigmoid(img) * w_row[:, 1:2]

    h = jnp.maximum(Wa1T_ref[...] * c1_ref[0] + col(tba_ref[pl.ds(b, 1), :]), 0.0)
    h = jnp.maximum(dot(Wa2T_ref[...], h) + col(ba2_ref[...]), 0.0)
    aud = jnp.tanh(dot(Wa3T_ref[...], h) + ba3_ref[...])
    aud_ref[0] = aud * w_row[:, 2:3]


def _full(shape):
    return pl.BlockSpec(shape, lambda b: (0,) * len(shape))


@jax.jit
def kernel(user_intent, coords_3d, coords_2d, coords_1d, W1, b1, W2, b2, Wr,
           br, Wt, bt, Wg1, bg1, Wg2, bg2, Wgs, bgs, Wgc, bgc, Wo1, bo1, Wo2,
           bo2, Wo3, bo3, Wa1, ba1, Wa2, ba2, Wa3, ba3):
    del Wgc, bgc
    B, N = _B, _N
    row = lambda v: v.reshape(1, -1)

    in_specs = [
        _full((B, 64)),
        pl.BlockSpec((1, 3, N), lambda b: (b, 0, 0)),
        pl.BlockSpec((1, 2, N), lambda b: (b, 0, 0)),
        pl.BlockSpec((1, 1, N), lambda b: (b, 0, 0)),
        _full((64, 64)), _full((1, 64)), _full((64, 64)), _full((1, 64)),
        _full((64, 3)), _full((1, 3)), _full((64, 16)), _full((1, 16)),
        _full((19, 256)), _full((1, 256)),
        _full((18, 256)), _full((1, 256)),
        _full((17, 256)), _full((1, 256)),
        _full((256, 256)), _full((1, 256)), _full((256, 1)), _full((1, 1)),
        _full((256, 256)), _full((1, 256)), _full((256, 3)), _full((1, 3)),
        _full((256, 256)), _full((1, 256)), _full((256, 1)), _full((1, 1)),
    ]
    out_specs = [
        _full((B, 3)),
        pl.BlockSpec((1, 1, N), lambda b: (b, 0, 0)),
        pl.BlockSpec((1, 3, N), lambda b: (b, 0, 0)),
        pl.BlockSpec((1, 1, N), lambda b: (b, 0, 0)),
    ]
    out_shapes = [
        jax.ShapeDtypeStruct((B, 3), jnp.float32),
        jax.ShapeDtypeStruct((B, 1, N), jnp.float32),
        jax.ShapeDtypeStruct((B, 3, N), jnp.float32),
        jax.ShapeDtypeStruct((B, 1, N), jnp.float32),
    ]

    weights, sdf_t, img_t, aud_t = pl.pallas_call(
        _body,
        grid=(B,),
        in_specs=in_specs,
        out_specs=out_specs,
        out_shape=out_shapes,
        scratch_shapes=[
            pltpu.VMEM((B, 256), jnp.float32),
            pltpu.VMEM((B, 256), jnp.float32),
            pltpu.VMEM((B, 256), jnp.float32),
            pltpu.VMEM((256, 3), jnp.float32),
            pltpu.VMEM((256, 2), jnp.float32),
            pltpu.VMEM((256, 1), jnp.float32),
            pltpu.VMEM((256, 256), jnp.float32),
            pltpu.VMEM((256, 256), jnp.float32),
            pltpu.VMEM((256, 256), jnp.float32),
            pltpu.VMEM((1, 256), jnp.float32),
            pltpu.VMEM((3, 256), jnp.float32),
            pltpu.VMEM((1, 256), jnp.float32),
        ],
    )(user_intent,
      coords_3d.transpose(0, 2, 1), coords_2d.transpose(0, 2, 1),
      coords_1d.transpose(0, 2, 1),
      W1, row(b1), W2, row(b2), Wr, row(br), Wt, row(bt),
      Wg1, row(bg1), Wo1, row(bo1), Wa1, row(ba1),
      Wg2, row(bg2), Wgs, row(bgs),
      Wo2, row(bo2), Wo3, row(bo3),
      Wa2, row(ba2), Wa3, row(ba3))

    raw_sdf = sdf_t.reshape(B, N, 1)
    raw_img = img_t.transpose(0, 2, 1)
    raw_audio = aud_t.reshape(B, N, 1)
    return weights, raw_sdf, raw_img, raw_audio
